# Initial kernel scaffold; baseline (speedup 1.0000x reference)
import jax
import jax.numpy as jnp
from jax.experimental import pallas as pl


def kernel(coords, lj_radius, lj_wdepth, pose_ind_for_rot, block_ind_for_rot, block_type_ind_for_rot):
    raise NotImplementedError("write your pallas kernel here")


# speedup vs baseline: 2.5989x; 2.5989x over previous
import functools

import jax
import jax.numpy as jnp
from jax import lax
from jax.experimental import pallas as pl
from jax.experimental.pallas import tpu as pltpu
from jax.experimental.pallas import tpu_sc as plsc

N_POSE_SLOTS = 16
N_TYPES = 20
NC = 2
NS = 16
LANES = 16


def _prep_body(c_ref, rad_ref, wd_ref, bt_ref, pose_ref,
               cx_ref, cy_ref, cz_ref, sig_ref, seps_ref, lo_ref, hi_ref):
    c = c_ref[...]
    cen = jnp.mean(c, axis=1)
    cx_ref[...] = cen[0:1, :]
    cy_ref[...] = cen[1:2, :]
    cz_ref[...] = cen[2:3, :]

    bt = bt_ref[...]
    sig = jnp.zeros(bt.shape, jnp.float32)
    seps = jnp.zeros(bt.shape, jnp.float32)
    for t in range(N_TYPES):
        sig = jnp.where(bt == t, rad_ref[t], sig)
        seps = jnp.where(bt == t, jnp.sqrt(wd_ref[t]), seps)
    sig_ref[...] = sig
    seps_ref[...] = seps

    pose = pose_ref[...]
    lo = jnp.zeros(pose.shape, jnp.int32)
    hi = jnp.zeros(pose.shape, jnp.int32)
    start = jnp.int32(0)
    for p in range(N_POSE_SLOTS):
        cnt = jnp.sum((pose == p).astype(jnp.int32))
        end = start + cnt
        lo = jnp.where(pose == p, start, lo)
        hi = jnp.where(pose == p, end, hi)
        start = end
    lo_ref[...] = lo
    hi_ref[...] = hi


def _prep_call(coords3, lj_radius, lj_wdepth, bt2, pose2):
    n = coords3.shape[-1]
    f = jax.ShapeDtypeStruct((1, n), jnp.float32)
    i = jax.ShapeDtypeStruct((1, n), jnp.int32)
    return pl.pallas_call(
        _prep_body,
        out_shape=[f, f, f, f, f, i, i],
        in_specs=[
            pl.BlockSpec(memory_space=pltpu.VMEM),
            pl.BlockSpec(memory_space=pltpu.SMEM),
            pl.BlockSpec(memory_space=pltpu.SMEM),
            pl.BlockSpec(memory_space=pltpu.VMEM),
            pl.BlockSpec(memory_space=pltpu.VMEM),
        ],
    )(coords3, lj_radius, lj_wdepth, bt2, pose2)


def _sc_body(n_rots, rows_per_w,
             x_hbm, y_hbm, z_hbm, sg_hbm, ep_hbm, lo_hbm, hi_hbm,
             pose_hbm, block_hbm, out_hbm,
             xv, yv, zv, sgv, epv, blkv, accv, lo_v, hi_v, po_v):
    wid = lax.axis_index("s") * NC + lax.axis_index("c")
    base = wid * rows_per_w

    pltpu.sync_copy(x_hbm, xv)
    pltpu.sync_copy(y_hbm, yv)
    pltpu.sync_copy(z_hbm, zv)
    pltpu.sync_copy(sg_hbm, sgv)
    pltpu.sync_copy(ep_hbm, epv)
    pltpu.sync_copy(block_hbm, blkv)
    pltpu.sync_copy(lo_hbm.at[pl.ds(base, rows_per_w)], lo_v)
    pltpu.sync_copy(hi_hbm.at[pl.ds(base, rows_per_w)], hi_v)
    pltpu.sync_copy(pose_hbm.at[pl.ds(base, rows_per_w)], po_v)

    lane_iota = lax.iota(jnp.int32, LANES)

    def _extract(vec, lane):
        z = jnp.where(lane_iota == lane, vec, jnp.zeros_like(vec))
        return jnp.sum(z)

    def row_body(r, pose_acc):
        i = base + r
        rg = jnp.bitwise_and(r, jnp.int32(-LANES))
        rl = jnp.bitwise_and(r, jnp.int32(LANES - 1))
        lo = _extract(lo_v[pl.ds(rg, LANES)], rl)
        hi = _extract(hi_v[pl.ds(rg, LANES)], rl)
        p = _extract(po_v[pl.ds(rg, LANES)], rl)
        isplat = jnp.full((LANES,), i, jnp.int32)
        xi = plsc.load_gather(xv, [isplat])
        yi = plsc.load_gather(yv, [isplat])
        zi = plsc.load_gather(zv, [isplat])
        si = plsc.load_gather(sgv, [isplat])
        ei = plsc.load_gather(epv, [isplat])
        bi = plsc.load_gather(blkv, [isplat])

        j0 = jnp.bitwise_and(lo, jnp.int32(-LANES))
        n_it = lax.shift_right_logical(hi - j0 + (LANES - 1), 4)

        def col_body(t, acc):
            js = j0 + t * LANES
            jvec = js + lane_iota
            xj = xv[pl.ds(js, LANES)]
            yj = yv[pl.ds(js, LANES)]
            zj = zv[pl.ds(js, LANES)]
            sj = sgv[pl.ds(js, LANES)]
            ej = epv[pl.ds(js, LANES)]
            bj = blkv[pl.ds(js, LANES)]
            dx = xi - xj
            dy = yi - yj
            dz = zi - zj
            d2 = jnp.maximum(dx * dx + dy * dy + dz * dz, jnp.float32(0.01))
            s = si + sj
            q = (s * s) / d2
            q3 = q * q * q
            lj = (ei * ej) * (q3 * q3 - 2.0 * q3)
            m = (jvec >= lo) & (jvec < hi) & (bj != bi)
            return acc + jnp.where(m, lj, jnp.float32(0.0))

        acc = lax.fori_loop(0, n_it, col_body, jnp.zeros((LANES,), jnp.float32))
        rs = jnp.sum(acc)
        return pose_acc + jnp.where(lane_iota == p, rs, jnp.float32(0.0))

    pose_acc = lax.fori_loop(0, rows_per_w, row_body,
                             jnp.zeros((LANES,), jnp.float32))
    accv[...] = pose_acc
    pltpu.sync_copy(accv, out_hbm.at[pl.ds(wid * LANES, LANES)])


def _sc_call(x, y, z, sg, ep, lo, hi, pose, block):
    n = pose.shape[0]
    nw = NC * NS
    rows_per_w = n // nw
    mesh = plsc.VectorSubcoreMesh(core_axis_name="c", subcore_axis_name="s",
                                  num_cores=NC, num_subcores=NS)
    kern = functools.partial(
        pl.kernel,
        out_type=jax.ShapeDtypeStruct((nw * LANES,), jnp.float32),
        mesh=mesh,
        compiler_params=pltpu.CompilerParams(needs_layout_passes=False),
        scratch_types=[
            pltpu.VMEM((n,), jnp.float32),
            pltpu.VMEM((n,), jnp.float32),
            pltpu.VMEM((n,), jnp.float32),
            pltpu.VMEM((n,), jnp.float32),
            pltpu.VMEM((n,), jnp.float32),
            pltpu.VMEM((n,), jnp.int32),
            pltpu.VMEM((LANES,), jnp.float32),
            pltpu.VMEM((rows_per_w,), jnp.int32),
            pltpu.VMEM((rows_per_w,), jnp.int32),
            pltpu.VMEM((rows_per_w,), jnp.int32),
        ],
    )(functools.partial(_sc_body, n, rows_per_w))
    return kern(x, y, z, sg, ep, lo, hi, pose, block)


def kernel(coords, lj_radius, lj_wdepth, pose_ind_for_rot, block_ind_for_rot,
           block_type_ind_for_rot):
    n = coords.shape[0]
    coords3 = coords.transpose(2, 1, 0)
    bt2 = block_type_ind_for_rot.reshape(1, n)
    pose2 = pose_ind_for_rot.reshape(1, n)
    cx, cy, cz, sig, seps, lo, hi = _prep_call(
        coords3, lj_radius, lj_wdepth, bt2, pose2)
    partials = _sc_call(cx.reshape(n), cy.reshape(n), cz.reshape(n),
                        sig.reshape(n), seps.reshape(n),
                        lo.reshape(n), hi.reshape(n),
                        pose_ind_for_rot, block_ind_for_rot)
    return 0.5 * jnp.sum(partials.reshape(NC * NS, LANES), axis=0)


# speedup vs baseline: 3.3903x; 1.3045x over previous
import functools

import jax
import jax.numpy as jnp
from jax import lax
from jax.experimental import pallas as pl
from jax.experimental.pallas import tpu as pltpu
from jax.experimental.pallas import tpu_sc as plsc

N_POSE_SLOTS = 16
N_TYPES = 20
NC = 2
NS = 16
LANES = 16
RBLK = 4


def _prep_body(c_ref, rad_ref, wd_ref, bt_ref, pose_ref,
               cx_ref, cy_ref, cz_ref, sig_ref, seps_ref, hi_ref):
    c = c_ref[...]
    cen = jnp.mean(c, axis=1)
    cx_ref[...] = cen[0:1, :]
    cy_ref[...] = cen[1:2, :]
    cz_ref[...] = cen[2:3, :]

    bt = bt_ref[...]
    sig = jnp.zeros(bt.shape, jnp.float32)
    seps = jnp.zeros(bt.shape, jnp.float32)
    for t in range(N_TYPES):
        sig = jnp.where(bt == t, rad_ref[t], sig)
        seps = jnp.where(bt == t, jnp.sqrt(wd_ref[t]), seps)
    sig_ref[...] = sig
    seps_ref[...] = seps

    pose = pose_ref[...]
    hi = jnp.zeros(pose.shape, jnp.int32)
    start = jnp.int32(0)
    for p in range(N_POSE_SLOTS):
        cnt = jnp.sum((pose == p).astype(jnp.int32))
        end = start + cnt
        hi = jnp.where(pose == p, end, hi)
        start = end
    hi_ref[...] = hi


def _prep_call(coords3, lj_radius, lj_wdepth, bt2, pose2):
    n = coords3.shape[-1]
    f = jax.ShapeDtypeStruct((1, n), jnp.float32)
    i = jax.ShapeDtypeStruct((1, n), jnp.int32)
    return pl.pallas_call(
        _prep_body,
        out_shape=[f, f, f, f, f, i],
        in_specs=[
            pl.BlockSpec(memory_space=pltpu.VMEM),
            pl.BlockSpec(memory_space=pltpu.SMEM),
            pl.BlockSpec(memory_space=pltpu.SMEM),
            pl.BlockSpec(memory_space=pltpu.VMEM),
            pl.BlockSpec(memory_space=pltpu.VMEM),
        ],
    )(coords3, lj_radius, lj_wdepth, bt2, pose2)


def _sc_body(n_rots, x_hbm, y_hbm, z_hbm, sg_hbm, ep_hbm, hi_hbm,
             pose_hbm, block_hbm, out_hbm,
             xv, yv, zv, sgv, epv, blkv, pov, hiv, accv):
    wid = lax.axis_index("s") * NC + lax.axis_index("c")
    n_blocks = n_rots // (RBLK * NC * NS)
    stride = RBLK * NC * NS

    pltpu.sync_copy(x_hbm, xv)
    pltpu.sync_copy(y_hbm, yv)
    pltpu.sync_copy(z_hbm, zv)
    pltpu.sync_copy(sg_hbm, sgv)
    pltpu.sync_copy(ep_hbm, epv)
    pltpu.sync_copy(block_hbm, blkv)
    pltpu.sync_copy(pose_hbm, pov)
    pltpu.sync_copy(hi_hbm, hiv)

    lane_iota = lax.iota(jnp.int32, LANES)
    zeros = jnp.zeros((LANES,), jnp.float32)
    for q in range(N_POSE_SLOTS):
        accv[pl.ds(q * LANES, LANES)] = zeros

    def blk_body(k, carry):
        i0 = wid * RBLK + k * stride
        last = i0 + (RBLK - 1)
        g0 = jnp.bitwise_and(last, jnp.int32(-LANES))
        hvec = hiv[pl.ds(g0, LANES)]
        hi_max = jnp.sum(jnp.where(lane_iota == (last - g0), hvec,
                                   jnp.zeros_like(hvec)))

        rows = []
        for r in range(RBLK):
            isplat = jnp.full((LANES,), i0 + r, jnp.int32)
            rows.append((
                plsc.load_gather(xv, [isplat]),
                plsc.load_gather(yv, [isplat]),
                plsc.load_gather(zv, [isplat]),
                plsc.load_gather(sgv, [isplat]),
                plsc.load_gather(epv, [isplat]),
                plsc.load_gather(blkv, [isplat]),
                plsc.load_gather(pov, [isplat]),
            ))

        jstart = jnp.bitwise_and(i0 + 1, jnp.int32(-LANES))
        n_it = lax.shift_right_arithmetic(hi_max - jstart + (LANES - 1), 4)

        def col_body(t, accs):
            js = jstart + t * LANES
            jvec = js + lane_iota
            xj = xv[pl.ds(js, LANES)]
            yj = yv[pl.ds(js, LANES)]
            zj = zv[pl.ds(js, LANES)]
            sj = sgv[pl.ds(js, LANES)]
            ej = epv[pl.ds(js, LANES)]
            bj = blkv[pl.ds(js, LANES)]
            pj = pov[pl.ds(js, LANES)]
            out = []
            for r in range(RBLK):
                xi, yi, zi, si, ei, bi, pi = rows[r]
                dx = xi - xj
                dy = yi - yj
                dz = zi - zj
                d2 = jnp.maximum(dx * dx + dy * dy + dz * dz,
                                 jnp.float32(0.01))
                s = si + sj
                q = (s * s) / d2
                q3 = q * q * q
                t6 = ej * (q3 * (q3 - 2.0))
                m = (pj == pi) & (bj != bi) & (jvec > (i0 + r))
                out.append(accs[r] + jnp.where(m, t6, jnp.float32(0.0)))
            return tuple(out)

        accs = lax.fori_loop(0, n_it, col_body,
                             tuple(zeros for _ in range(RBLK)))
        for r in range(RBLK):
            _, _, _, _, ei, _, pi = rows[r]
            idx = pi * LANES + lane_iota
            plsc.addupdate_scatter(accv, [idx], ei * accs[r])
        return carry

    lax.fori_loop(0, n_blocks, blk_body, jnp.int32(0))
    pltpu.sync_copy(accv, out_hbm.at[pl.ds(wid * (N_POSE_SLOTS * LANES),
                                           N_POSE_SLOTS * LANES)])


def _sc_call(x, y, z, sg, ep, hi, pose, block):
    n = pose.shape[0]
    nw = NC * NS
    mesh = plsc.VectorSubcoreMesh(core_axis_name="c", subcore_axis_name="s",
                                  num_cores=NC, num_subcores=NS)
    kern = functools.partial(
        pl.kernel,
        out_type=jax.ShapeDtypeStruct((nw * N_POSE_SLOTS * LANES,),
                                      jnp.float32),
        mesh=mesh,
        compiler_params=pltpu.CompilerParams(needs_layout_passes=False),
        scratch_types=[
            pltpu.VMEM((n,), jnp.float32),
            pltpu.VMEM((n,), jnp.float32),
            pltpu.VMEM((n,), jnp.float32),
            pltpu.VMEM((n,), jnp.float32),
            pltpu.VMEM((n,), jnp.float32),
            pltpu.VMEM((n,), jnp.int32),
            pltpu.VMEM((n,), jnp.int32),
            pltpu.VMEM((n,), jnp.int32),
            pltpu.VMEM((N_POSE_SLOTS * LANES,), jnp.float32),
        ],
    )(functools.partial(_sc_body, n))
    return kern(x, y, z, sg, ep, hi, pose, block)


def kernel(coords, lj_radius, lj_wdepth, pose_ind_for_rot, block_ind_for_rot,
           block_type_ind_for_rot):
    n = coords.shape[0]
    coords3 = coords.transpose(2, 1, 0)
    bt2 = block_type_ind_for_rot.reshape(1, n)
    pose2 = pose_ind_for_rot.reshape(1, n)
    cx, cy, cz, sig, seps, hi = _prep_call(
        coords3, lj_radius, lj_wdepth, bt2, pose2)
    partials = _sc_call(cx.reshape(n), cy.reshape(n), cz.reshape(n),
                        sig.reshape(n), seps.reshape(n), hi.reshape(n),
                        pose_ind_for_rot, block_ind_for_rot)
    return jnp.sum(partials.reshape(NC * NS, N_POSE_SLOTS, LANES),
                   axis=(0, 2))
